# baseline (speedup 1.0000x reference)
import jax
import jax.numpy as jnp
from jax.experimental import pallas as pl

_R = 4


def _dup_linear_kernel(x_ref, wt_ref, b_ref, o_ref):
    y = jnp.dot(x_ref[...], wt_ref[...], preferred_element_type=jnp.float32)
    y = y + b_ref[...]
    o_ref[...] = jnp.concatenate([y] * _R, axis=1)


def kernel(x, edge_index, W, b):
    n, c_in = x.shape
    c_out = W.shape[0]
    wt = W.T.astype(jnp.float32)
    b2 = b.reshape(1, c_out).astype(jnp.float32)

    bn = 1000
    grid = (n // bn,)
    out = pl.pallas_call(
        _dup_linear_kernel,
        grid=grid,
        in_specs=[
            pl.BlockSpec((bn, c_in), lambda i: (i, i - i)),
            pl.BlockSpec((c_in, c_out), lambda i: (i - i, i - i)),
            pl.BlockSpec((1, c_out), lambda i: (i - i, i - i)),
        ],
        out_specs=pl.BlockSpec((bn, _R * c_out), lambda i: (i, i - i)),
        out_shape=jax.ShapeDtypeStruct((n, _R * c_out), jnp.float32),
    )(x.astype(jnp.float32), wt, b2)
    return out.reshape(n * _R, c_out)


# speedup vs baseline: 2.4655x; 2.4655x over previous
import jax
import jax.numpy as jnp
from jax.experimental import pallas as pl

_R = 4


def _dup_linear_kernel(x_ref, wt_ref, b_ref, o_ref):
    y = jnp.dot(x_ref[...], wt_ref[...], preferred_element_type=jnp.float32)
    y = y + b_ref[...]
    o_ref[...] = jnp.repeat(y, _R, axis=0)


def kernel(x, edge_index, W, b):
    n, c_in = x.shape
    c_out = W.shape[0]
    wt = W.T.astype(jnp.float32)
    b2 = b.reshape(1, c_out).astype(jnp.float32)

    bn = 1000
    grid = (n // bn,)
    out = pl.pallas_call(
        _dup_linear_kernel,
        grid=grid,
        in_specs=[
            pl.BlockSpec((bn, c_in), lambda i: (i, i - i)),
            pl.BlockSpec((c_in, c_out), lambda i: (i - i, i - i)),
            pl.BlockSpec((1, c_out), lambda i: (i - i, i - i)),
        ],
        out_specs=pl.BlockSpec((_R * bn, c_out), lambda i: (i, i - i)),
        out_shape=jax.ShapeDtypeStruct((_R * n, c_out), jnp.float32),
    )(x.astype(jnp.float32), wt, b2)
    return out


# speedup vs baseline: 2.6897x; 1.0910x over previous
import jax
import jax.numpy as jnp
from jax.experimental import pallas as pl

_R = 4


def _dup_linear_kernel(x_ref, wt_ref, b_ref, o_ref):
    y = jnp.dot(x_ref[...], wt_ref[...], preferred_element_type=jnp.float32)
    y = y + b_ref[...]
    for r in range(_R):
        o_ref[r::_R, :] = y


def kernel(x, edge_index, W, b):
    n, c_in = x.shape
    c_out = W.shape[0]
    wt = W.T.astype(jnp.float32)
    b2 = b.reshape(1, c_out).astype(jnp.float32)

    bn = 1000
    grid = (n // bn,)
    out = pl.pallas_call(
        _dup_linear_kernel,
        grid=grid,
        in_specs=[
            pl.BlockSpec((bn, c_in), lambda i: (i, i - i)),
            pl.BlockSpec((c_in, c_out), lambda i: (i - i, i - i)),
            pl.BlockSpec((1, c_out), lambda i: (i - i, i - i)),
        ],
        out_specs=pl.BlockSpec((_R * bn, c_out), lambda i: (i, i - i)),
        out_shape=jax.ShapeDtypeStruct((_R * n, c_out), jnp.float32),
    )(x.astype(jnp.float32), wt, b2)
    return out


# speedup vs baseline: 2.6927x; 1.0011x over previous
import jax
import jax.numpy as jnp
from jax.experimental import pallas as pl
from jax.experimental.pallas import tpu as pltpu

_R = 4


def _dup_linear_kernel(x_ref, wt_ref, b_ref, o_ref):
    y = jnp.dot(x_ref[...], wt_ref[...], preferred_element_type=jnp.float32)
    y = y + b_ref[...]
    for r in range(_R):
        o_ref[r::_R, :] = y


def kernel(x, edge_index, W, b):
    n, c_in = x.shape
    c_out = W.shape[0]
    wt = W.T.astype(jnp.float32)
    b2 = b.reshape(1, c_out).astype(jnp.float32)

    bn = 1000
    grid = (n // bn,)
    out = pl.pallas_call(
        _dup_linear_kernel,
        grid=grid,
        in_specs=[
            pl.BlockSpec((bn, c_in), lambda i: (i, i - i)),
            pl.BlockSpec((c_in, c_out), lambda i: (i - i, i - i)),
            pl.BlockSpec((1, c_out), lambda i: (i - i, i - i)),
        ],
        out_specs=pl.BlockSpec((_R * bn, c_out), lambda i: (i, i - i)),
        out_shape=jax.ShapeDtypeStruct((_R * n, c_out), jnp.float32),
        compiler_params=pltpu.CompilerParams(
            dimension_semantics=("parallel",)),
    )(x.astype(jnp.float32), wt, b2)
    return out


# speedup vs baseline: 3.5079x; 1.3027x over previous
import jax
import jax.numpy as jnp
from jax.experimental import pallas as pl
from jax.experimental.pallas import tpu as pltpu

_R = 4


def _dup_linear_kernel(x_ref, wt_ref, b_ref, o_ref):
    y = jnp.dot(x_ref[...], wt_ref[...], preferred_element_type=jnp.float32)
    y = y + b_ref[...]
    for r in range(_R):
        o_ref[r::_R, :] = y


def kernel(x, edge_index, W, b):
    n, c_in = x.shape
    c_out = W.shape[0]
    wt = W.T.astype(jnp.float32)
    b2 = b.reshape(1, c_out).astype(jnp.float32)

    bn = 2000
    grid = (n // bn,)
    out = pl.pallas_call(
        _dup_linear_kernel,
        grid=grid,
        in_specs=[
            pl.BlockSpec((bn, c_in), lambda i: (i, i - i)),
            pl.BlockSpec((c_in, c_out), lambda i: (i - i, i - i)),
            pl.BlockSpec((1, c_out), lambda i: (i - i, i - i)),
        ],
        out_specs=pl.BlockSpec((_R * bn, c_out), lambda i: (i, i - i)),
        out_shape=jax.ShapeDtypeStruct((_R * n, c_out), jnp.float32),
        compiler_params=pltpu.CompilerParams(
            dimension_semantics=("parallel",)),
    )(x.astype(jnp.float32), wt, b2)
    return out


# speedup vs baseline: 3.8149x; 1.0875x over previous
import jax
import jax.numpy as jnp
from jax.experimental import pallas as pl
from jax.experimental.pallas import tpu as pltpu

_R = 4


def _dup_linear_kernel(x_ref, wt_ref, b_ref, o_ref):
    y = jnp.dot(x_ref[...], wt_ref[...], preferred_element_type=jnp.float32)
    y = y + b_ref[...]
    for r in range(_R):
        o_ref[r::_R, :] = y


def kernel(x, edge_index, W, b):
    n, c_in = x.shape
    c_out = W.shape[0]
    wt = W.T.astype(jnp.float32)
    b2 = b.reshape(1, c_out).astype(jnp.float32)

    bn = 5000
    grid = (n // bn,)
    out = pl.pallas_call(
        _dup_linear_kernel,
        grid=grid,
        in_specs=[
            pl.BlockSpec((bn, c_in), lambda i: (i, i - i)),
            pl.BlockSpec((c_in, c_out), lambda i: (i - i, i - i)),
            pl.BlockSpec((1, c_out), lambda i: (i - i, i - i)),
        ],
        out_specs=pl.BlockSpec((_R * bn, c_out), lambda i: (i, i - i)),
        out_shape=jax.ShapeDtypeStruct((_R * n, c_out), jnp.float32),
        compiler_params=pltpu.CompilerParams(
            dimension_semantics=("parallel",)),
    )(x.astype(jnp.float32), wt, b2)
    return out


# speedup vs baseline: 3.9074x; 1.0242x over previous
import jax
import jax.numpy as jnp
from jax.experimental import pallas as pl
from jax.experimental.pallas import tpu as pltpu

_R = 4


def _dup_linear_kernel(x_ref, wt_ref, b_ref, o_ref):
    y = jnp.dot(x_ref[...], wt_ref[...], preferred_element_type=jnp.float32)
    y = y + b_ref[...]
    for r in range(_R):
        o_ref[r::_R, :] = y


def kernel(x, edge_index, W, b):
    n, c_in = x.shape
    c_out = W.shape[0]
    wt = W.T.astype(jnp.float32)
    b2 = b.reshape(1, c_out).astype(jnp.float32)

    bn = 10000
    grid = (n // bn,)
    out = pl.pallas_call(
        _dup_linear_kernel,
        grid=grid,
        in_specs=[
            pl.BlockSpec((bn, c_in), lambda i: (i, i - i)),
            pl.BlockSpec((c_in, c_out), lambda i: (i - i, i - i)),
            pl.BlockSpec((1, c_out), lambda i: (i - i, i - i)),
        ],
        out_specs=pl.BlockSpec((_R * bn, c_out), lambda i: (i, i - i)),
        out_shape=jax.ShapeDtypeStruct((_R * n, c_out), jnp.float32),
        compiler_params=pltpu.CompilerParams(
            dimension_semantics=("parallel",)),
    )(x.astype(jnp.float32), wt, b2)
    return out


# speedup vs baseline: 4.0633x; 1.0399x over previous
import jax
import jax.numpy as jnp
from jax.experimental import pallas as pl
from jax.experimental.pallas import tpu as pltpu

_R = 4


def _dup_linear_kernel(x_ref, w_ref, b_ref, o_ref):
    y = jax.lax.dot_general(
        x_ref[...], w_ref[...], (((1,), (1,)), ((), ())),
        preferred_element_type=jnp.float32)
    y = y + b_ref[...]
    for r in range(_R):
        o_ref[r::_R, :] = y


def kernel(x, edge_index, W, b):
    n, c_in = x.shape
    c_out = W.shape[0]
    b2 = b.reshape(1, c_out)

    bn = 10000
    grid = (n // bn,)
    out = pl.pallas_call(
        _dup_linear_kernel,
        grid=grid,
        in_specs=[
            pl.BlockSpec((bn, c_in), lambda i: (i, i - i)),
            pl.BlockSpec((c_out, c_in), lambda i: (i - i, i - i)),
            pl.BlockSpec((1, c_out), lambda i: (i - i, i - i)),
        ],
        out_specs=pl.BlockSpec((_R * bn, c_out), lambda i: (i, i - i)),
        out_shape=jax.ShapeDtypeStruct((_R * n, c_out), jnp.float32),
        compiler_params=pltpu.CompilerParams(
            dimension_semantics=("parallel",)),
    )(x, W, b2)
    return out
